# baseline (speedup 1.0000x reference)
import jax
import jax.numpy as jnp
from jax.experimental import pallas as pl
from jax.experimental.pallas import tpu as pltpu

_TM = 400
_NBUF = 3


def _pipelined_kernel(adj_hbm, x_ref, w_ref, b_ref, o_ref, buf, sems):
    n = o_ref.shape[0]
    nblk = n // _TM
    x = x_ref[...]
    wt = w_ref[...]
    bias = b_ref[...]

    def start(blk, slot):
        pltpu.make_async_copy(
            adj_hbm.at[pl.ds(blk * _TM, _TM), :], buf.at[slot], sems.at[slot]
        ).start()

    for s in range(_NBUF):
        start(s, s)

    def body(i, carry):
        slot = jax.lax.rem(i, _NBUF)
        pltpu.make_async_copy(
            adj_hbm.at[pl.ds(i * _TM, _TM), :], buf.at[slot], sems.at[slot]
        ).wait()
        agg = jnp.dot(buf[slot], x, preferred_element_type=jnp.float32)
        y = jax.lax.dot_general(
            agg, wt, (((1,), (1,)), ((), ())),
            preferred_element_type=jnp.float32,
        )
        o_ref[pl.ds(i * _TM, _TM), :] = jnp.tanh(y + bias)

        @pl.when(i + _NBUF < nblk)
        def _():
            start(i + _NBUF, slot)

        return carry

    jax.lax.fori_loop(0, nblk, body, 0)


def kernel(input, adj, W, b):
    n, k = adj.shape
    _, d = input.shape
    b2 = b.reshape(1, d)
    return pl.pallas_call(
        _pipelined_kernel,
        in_specs=[
            pl.BlockSpec(memory_space=pltpu.HBM),
            pl.BlockSpec(memory_space=pltpu.VMEM),
            pl.BlockSpec(memory_space=pltpu.VMEM),
            pl.BlockSpec(memory_space=pltpu.VMEM),
        ],
        out_specs=pl.BlockSpec(memory_space=pltpu.VMEM),
        out_shape=jax.ShapeDtypeStruct((n, d), jnp.float32),
        compiler_params=pltpu.CompilerParams(
            vmem_limit_bytes=64 * 1024 * 1024,
        ),
        scratch_shapes=[
            pltpu.VMEM((_NBUF, _TM, k), jnp.float32),
            pltpu.SemaphoreType.DMA((_NBUF,)),
        ],
    )(adj, input, W, b2)


# speedup vs baseline: 1.0322x; 1.0322x over previous
import jax
import jax.numpy as jnp
from jax.experimental import pallas as pl
from jax.experimental.pallas import tpu as pltpu

_TM = 480


def _fused_gcn_kernel(adj_ref, x_ref, w_ref, b_ref, o_ref):
    agg = jnp.dot(adj_ref[...], x_ref[...], preferred_element_type=jnp.float32)
    y = jax.lax.dot_general(
        agg, w_ref[...], (((1,), (1,)), ((), ())),
        preferred_element_type=jnp.float32,
    )
    o_ref[...] = jnp.tanh(y + b_ref[...])


def kernel(input, adj, W, b):
    n, k = adj.shape
    _, d = input.shape
    b2 = b.reshape(1, d)
    grid = (pl.cdiv(n, _TM),)
    return pl.pallas_call(
        _fused_gcn_kernel,
        grid=grid,
        in_specs=[
            pl.BlockSpec((_TM, k), lambda i: (i, 0)),
            pl.BlockSpec((k, d), lambda i: (0, 0)),
            pl.BlockSpec((d, d), lambda i: (0, 0)),
            pl.BlockSpec((1, d), lambda i: (0, 0)),
        ],
        out_specs=pl.BlockSpec((_TM, d), lambda i: (i, 0)),
        out_shape=jax.ShapeDtypeStruct((n, d), jnp.float32),
        compiler_params=pltpu.CompilerParams(
            dimension_semantics=("parallel",),
        ),
    )(adj, input, W, b2)


# speedup vs baseline: 1.0418x; 1.0093x over previous
import jax
import jax.numpy as jnp
from jax.experimental import pallas as pl
from jax.experimental.pallas import tpu as pltpu

_TM = 400


def _fused_gcn_kernel(adj_ref, x_ref, w_ref, b_ref, o_ref):
    agg = jnp.dot(adj_ref[...], x_ref[...], preferred_element_type=jnp.float32)
    y = jax.lax.dot_general(
        agg, w_ref[...], (((1,), (1,)), ((), ())),
        preferred_element_type=jnp.float32,
    )
    o_ref[...] = jnp.tanh(y + b_ref[...])


def kernel(input, adj, W, b):
    n, k = adj.shape
    _, d = input.shape
    b2 = b.reshape(1, d)
    grid = (n // _TM,)
    return pl.pallas_call(
        _fused_gcn_kernel,
        grid=grid,
        in_specs=[
            pl.BlockSpec((_TM, k), lambda i: (i, 0)),
            pl.BlockSpec((k, d), lambda i: (0, 0)),
            pl.BlockSpec((d, d), lambda i: (0, 0)),
            pl.BlockSpec((1, d), lambda i: (0, 0)),
        ],
        out_specs=pl.BlockSpec((_TM, d), lambda i: (i, 0)),
        out_shape=jax.ShapeDtypeStruct((n, d), jnp.float32),
        compiler_params=pltpu.CompilerParams(
            dimension_semantics=("arbitrary",),
        ),
    )(adj, input, W, b2)
